# Initial kernel scaffold; baseline (speedup 1.0000x reference)
import jax
import jax.numpy as jnp
from jax.experimental import pallas as pl


def kernel(xyz, feats, Wqkv, Wout, bout, ln_g, ln_b, W1, b1, W2, b2):
    raise NotImplementedError("write your pallas kernel here")


# speedup vs baseline: 2.7549x; 2.7549x over previous
import functools

import numpy as np
import jax
import jax.numpy as jnp
from jax import lax
from jax.experimental import pallas as pl
from jax.experimental.pallas import tpu as pltpu
from jax.experimental.pallas import tpu_sc as plsc

DIM = 256
NUM_HEADS = 8
DH = DIM // NUM_HEADS
RADIUS = 0.3
MAX_K = 32
N = 10000
M = 2500

NPAD = 10240
MPAD = 2560
CB = 256
PB = 512
NBLK_I = MPAD // CB
NBLK_J = NPAD // PB
BMROWS = MPAD // 32
NVEC = N // 16

NTILES = 32
EROWS = NPAD // NTILES

_BIG_COORD = np.float32(1.0e9)


def _center_indices():
    perm = jax.random.permutation(jax.random.key(1234), N)
    return perm[:M].astype(jnp.int32)


def _geom_body(ctr_ref, pts_ref, bm_ref, near_ref, gc_ref, cnt_ref,
               minv, mini, run):
    i = pl.program_id(0)
    j = pl.program_id(1)
    cx = ctr_ref[:, 0:1]
    cy = ctr_ref[:, 1:2]
    cz = ctr_ref[:, 2:3]
    px = pts_ref[0:1, :]
    py = pts_ref[1:2, :]
    pz = pts_ref[2:3, :]
    dx = cx - px
    dy = cy - py
    dz = cz - pz
    d2 = (dx * dx + dy * dy) + dz * dz
    dist = jnp.sqrt(jnp.maximum(d2, 0.0))
    mask = dist < jnp.float32(RADIUS)

    rowid = lax.broadcasted_iota(jnp.int32, (CB, PB), 0)
    bits = jnp.where(mask, jnp.left_shift(jnp.int32(1), rowid & 31), 0)
    bm_ref[...] = jnp.sum(bits.reshape(CB // 32, 32, PB), axis=1)

    maskf = mask.astype(jnp.float32)
    gsel = (
        lax.broadcasted_iota(jnp.int32, (PB, PB // 16), 0) >> 4
        == lax.broadcasted_iota(jnp.int32, (PB, PB // 16), 1)
    ).astype(jnp.float32)
    gcounts = jnp.dot(maskf, gsel, preferred_element_type=jnp.float32)
    gc_ref[0] = gcounts.astype(jnp.int32)

    tile_cnt = jnp.sum(maskf, axis=1, keepdims=True)
    old = jnp.where(j == 0, 0.0, run[...])
    newrun = old + tile_cnt
    run[...] = newrun

    sl = pl.ds(j * PB, PB)
    tmin = jnp.min(dist, axis=0, keepdims=True)
    rid_g = rowid + i * CB
    tidx = jnp.min(
        jnp.where(dist == tmin, rid_g, jnp.int32(1 << 30)),
        axis=0, keepdims=True,
    )

    @pl.when(i == 0)
    def _():
        minv[0:1, sl] = tmin
        mini[0:1, sl] = tidx

    @pl.when(i > 0)
    def _():
        upd = tmin < minv[0:1, sl]
        mini[0:1, sl] = jnp.where(upd, tidx, mini[0:1, sl])
        minv[0:1, sl] = jnp.where(upd, tmin, minv[0:1, sl])

    @pl.when(i == NBLK_I - 1)
    def _():
        near_ref[...] = jnp.broadcast_to(mini[0:1, sl], (8, PB))

    @pl.when(j == NBLK_J - 1)
    def _():
        cnt = jnp.minimum(newrun, np.float32(MAX_K)).astype(jnp.int32)
        cnt_ref[...] = jnp.broadcast_to(cnt, (CB, 128))


def _geom(ctr, pts, *, interpret=False):
    return pl.pallas_call(
        _geom_body,
        grid=(NBLK_I, NBLK_J),
        in_specs=[
            pl.BlockSpec((CB, 8), lambda i, j: (i, 0)),
            pl.BlockSpec((8, PB), lambda i, j: (0, j)),
        ],
        out_specs=[
            pl.BlockSpec((CB // 32, PB), lambda i, j: (i, j)),
            pl.BlockSpec((8, PB), lambda i, j: (0, j)),
            pl.BlockSpec((1, CB, PB // 16), lambda i, j: (j, i, 0)),
            pl.BlockSpec((CB, 128), lambda i, j: (i, 0)),
        ],
        out_shape=[
            jax.ShapeDtypeStruct((BMROWS, NPAD), jnp.int32),
            jax.ShapeDtypeStruct((8, NPAD), jnp.int32),
            jax.ShapeDtypeStruct((NBLK_J, MPAD, PB // 16), jnp.int32),
            jax.ShapeDtypeStruct((MPAD, 128), jnp.int32),
        ],
        scratch_shapes=[
            pltpu.VMEM((8, NPAD), jnp.float32),
            pltpu.VMEM((8, NPAD), jnp.int32),
            pltpu.VMEM((CB, 1), jnp.float32),
        ],
        interpret=interpret,
    )(ctr, pts)


def _kvproj_body(f_ref, w_ref, o_ref):
    o_ref[...] = jnp.dot(f_ref[...], w_ref[...],
                         preferred_element_type=jnp.float32)


def _kvproj(feats, wkv_t, *, interpret=False):
    rb = 1000
    return pl.pallas_call(
        _kvproj_body,
        grid=(N // rb,),
        in_specs=[
            pl.BlockSpec((rb, DIM), lambda i: (i, 0)),
            pl.BlockSpec((DIM, 2 * DIM), lambda i: (0, 0)),
        ],
        out_specs=pl.BlockSpec((rb, 2 * DIM), lambda i: (i, 0)),
        out_shape=jax.ShapeDtypeStruct((N, 2 * DIM), jnp.float32),
        interpret=interpret,
    )(feats, wkv_t)


def _compact_body(bm_hbm, gc_hbm, kv_hbm, feats_hbm, idxc_hbm,
                  nbrkv_hbm, cfeat_hbm,
                  bmrow, gcslab, idxbuf, idxout, kvbuf, icvec, cfbuf,
                  nsm, sem):
    wid = lax.axis_index("s") * 2 + lax.axis_index("c")
    zeros16 = jnp.zeros((16,), jnp.int32)
    iota16 = lax.iota(jnp.int32, 16)

    for t in range(3):
        r = wid + 32 * t

        @pl.when(r < BMROWS)
        def _():
            pltpu.sync_copy(bm_hbm.at[r], bmrow)
            for ch in range(NBLK_J):
                pltpu.sync_copy(gc_hbm.at[ch, pl.ds(r * 32, 32)],
                                gcslab.at[ch])
            pltpu.sync_copy(idxc_hbm.at[pl.ds(r * 32, 32)], icvec)
            pltpu.async_copy(feats_hbm.at[icvec], cfbuf, sem).wait()
            pltpu.sync_copy(cfbuf, cfeat_hbm.at[pl.ds(r * 32, 32)])

            def center_body(ci, _):
                c = r * 32 + ci

                @pl.when(c < M)
                def _():
                    idxbuf[pl.ds(0, 16)] = zeros16
                    idxbuf[pl.ds(16, 16)] = zeros16
                    idxbuf[pl.ds(32, 16)] = zeros16
                    nsm[0] = jnp.int32(0)

                    def chunk_body(ch, _):
                        n0 = nsm[0]

                        @pl.when(n0 < MAX_K)
                        def _():
                            g1 = gcslab[ch, ci, pl.ds(0, 16)]
                            g2 = gcslab[ch, ci, pl.ds(16, 16)]
                            n = n0
                            for k in range(32):
                                v = ch * 32 + k
                                words = bmrow[pl.ds(v * 16, 16)]
                                mvec = (jnp.right_shift(words, ci) & 1) == 1
                                vals = iota16 + v * 16
                                off = jnp.minimum(n, MAX_K)
                                plsc.store_compressed(
                                    idxbuf.at[pl.ds(off, 16)], vals,
                                    mask=mvec)
                                gk = g1[k] if k < 16 else g2[k - 16]
                                n = n + gk
                            nsm[0] = n
                        return ()

                    lax.fori_loop(0, NBLK_J, chunk_body, ())

                    idxout[pl.ds(0, 16)] = idxbuf[pl.ds(0, 16)]
                    idxout[pl.ds(16, 16)] = idxbuf[pl.ds(16, 16)]
                    pltpu.async_copy(kv_hbm.at[idxout], kvbuf, sem).wait()
                    pltpu.sync_copy(kvbuf, nbrkv_hbm.at[c])
                return ()

            lax.fori_loop(0, 32, center_body, ())


def _compact_gather(bitmap, gcounts, kv_all, feats, idxc_pad):
    mesh = plsc.VectorSubcoreMesh(core_axis_name="c", subcore_axis_name="s")
    f = functools.partial(
        pl.kernel,
        mesh=mesh,
        compiler_params=pltpu.CompilerParams(needs_layout_passes=False),
        out_type=[
            jax.ShapeDtypeStruct((MPAD, MAX_K, 2 * DIM), jnp.float32),
            jax.ShapeDtypeStruct((MPAD, DIM), jnp.float32),
        ],
        scratch_types=[
            pltpu.VMEM((NPAD,), jnp.int32),
            pltpu.VMEM((NBLK_J, 32, PB // 16), jnp.int32),
            pltpu.VMEM((48,), jnp.int32),
            pltpu.VMEM((MAX_K,), jnp.int32),
            pltpu.VMEM((MAX_K, 2 * DIM), jnp.float32),
            pltpu.VMEM((32,), jnp.int32),
            pltpu.VMEM((32, DIM), jnp.float32),
            pltpu.SMEM((1,), jnp.int32),
            pltpu.SemaphoreType.DMA,
        ],
    )(_compact_body)
    return f(bitmap, gcounts, kv_all, feats, idxc_pad)


def _ln(x, g, b):
    mu = jnp.mean(x, axis=-1, keepdims=True)
    var = jnp.mean((x - mu) ** 2, axis=-1, keepdims=True)
    return (x - mu) / jnp.sqrt(var + 1e-5) * g + b


CB2 = 128


def _attn_body(cf_ref, kv_ref, cnt_ref, wq_ref, wo_ref, bo_ref,
               g_ref, b_ref, w1_ref, b1_ref, w2_ref, b2_ref, out_ref):
    scale = np.float32(DH ** -0.5)
    cf0 = cf_ref[...]
    q = jnp.dot(cf0, wq_ref[...], preferred_element_type=jnp.float32) * scale
    cnt = cnt_ref[:, 0:1]
    valid = lax.broadcasted_iota(jnp.int32, (CB2, MAX_K), 1) < cnt

    outs = []
    for h in range(NUM_HEADS):
        kh = kv_ref[:, :, h * DH:(h + 1) * DH]
        vh = kv_ref[:, :, DIM + h * DH:DIM + (h + 1) * DH]
        qh = q[:, None, h * DH:(h + 1) * DH]
        s = jnp.sum(qh * kh, axis=-1)
        s = jnp.where(valid, s, jnp.float32(-1e9))
        m = jnp.max(s, axis=-1, keepdims=True)
        e = jnp.exp(s - m)
        a = e / jnp.sum(e, axis=-1, keepdims=True)
        outs.append(jnp.sum(a[:, :, None] * vh, axis=1))
    att = jnp.concatenate(outs, axis=-1)

    upd = jnp.dot(att, wo_ref[...],
                  preferred_element_type=jnp.float32) + bo_ref[...]
    g = g_ref[...]
    b = b_ref[...]
    cf1 = cf0 + _ln(upd, g, b)
    h1 = jnp.maximum(
        jnp.dot(cf1, w1_ref[...], preferred_element_type=jnp.float32)
        + b1_ref[...], 0.0)
    f2 = jnp.dot(h1, w2_ref[...],
                 preferred_element_type=jnp.float32) + b2_ref[...]
    out_ref[...] = cf1 + _ln(f2, g, b)


def _attn_ffn(cfeat, nbr_kv, cnt_b, wq_t, wo_t, bout, ln_g, ln_b,
              w1_t, b1, w2_t, b2, *, interpret=False):
    return pl.pallas_call(
        _attn_body,
        grid=(MPAD // CB2,),
        in_specs=[
            pl.BlockSpec((CB2, DIM), lambda i: (i, 0)),
            pl.BlockSpec((CB2, MAX_K, 2 * DIM), lambda i: (i, 0, 0)),
            pl.BlockSpec((CB2, 128), lambda i: (i, 0)),
            pl.BlockSpec((DIM, DIM), lambda i: (0, 0)),
            pl.BlockSpec((DIM, DIM), lambda i: (0, 0)),
            pl.BlockSpec((1, DIM), lambda i: (0, 0)),
            pl.BlockSpec((1, DIM), lambda i: (0, 0)),
            pl.BlockSpec((1, DIM), lambda i: (0, 0)),
            pl.BlockSpec((DIM, 4 * DIM), lambda i: (0, 0)),
            pl.BlockSpec((1, 4 * DIM), lambda i: (0, 0)),
            pl.BlockSpec((4 * DIM, DIM), lambda i: (0, 0)),
            pl.BlockSpec((1, DIM), lambda i: (0, 0)),
        ],
        out_specs=pl.BlockSpec((CB2, DIM), lambda i: (i, 0)),
        out_shape=jax.ShapeDtypeStruct((MPAD, DIM), jnp.float32),
        interpret=interpret,
    )(cfeat, nbr_kv, cnt_b, wq_t, wo_t, bout, ln_g, ln_b, w1_t, b1, w2_t, b2)


def _ups_body(cf_hbm, feats_hbm, near_hbm, out_hbm,
              idxv, cbuf, fbuf, obuf, sem):
    wid = lax.axis_index("s") * 2 + lax.axis_index("c")
    base = wid * EROWS

    def chunk_body(t, _):
        start = base + t * 32
        pltpu.sync_copy(near_hbm.at[pl.ds(start, 32)], idxv)
        cp = pltpu.async_copy(cf_hbm.at[idxv], cbuf, sem)
        pltpu.sync_copy(feats_hbm.at[pl.ds(start, 32)], fbuf)
        cp.wait()

        def row_body(i, _):
            def lane_body(kk, _):
                sl = pl.ds(kk * 16, 16)
                obuf[i, sl] = fbuf[i, sl] + cbuf[i, sl]
                return ()
            lax.fori_loop(0, DIM // 16, lane_body, ())
            return ()

        lax.fori_loop(0, 32, row_body, ())
        pltpu.sync_copy(obuf, out_hbm.at[pl.ds(start, 32)])
        return ()

    lax.fori_loop(0, EROWS // 32, chunk_body, ())


def _upsample(cf, feats_pad, nearest):
    mesh = plsc.VectorSubcoreMesh(core_axis_name="c", subcore_axis_name="s")
    f = functools.partial(
        pl.kernel,
        mesh=mesh,
        compiler_params=pltpu.CompilerParams(needs_layout_passes=False),
        out_type=jax.ShapeDtypeStruct((NPAD, DIM), jnp.float32),
        scratch_types=[
            pltpu.VMEM((32,), jnp.int32),
            pltpu.VMEM((32, DIM), jnp.float32),
            pltpu.VMEM((32, DIM), jnp.float32),
            pltpu.VMEM((32, DIM), jnp.float32),
            pltpu.SemaphoreType.DMA,
        ],
    )(_ups_body)
    return f(cf, feats_pad, nearest)


def kernel(xyz, feats, Wqkv, Wout, bout, ln_g, ln_b, W1, b1, W2, b2):
    idxc = _center_indices()
    idxc_pad = jnp.zeros((MPAD,), jnp.int32).at[:M].set(idxc)

    center_xyz = xyz[idxc]
    ctr = jnp.full((MPAD, 8), _BIG_COORD, jnp.float32)
    ctr = ctr.at[:M, 0:3].set(center_xyz)
    pts = jnp.full((8, NPAD), _BIG_COORD, jnp.float32)
    pts = pts.at[0:3, :N].set(xyz.T)

    bitmap, near8, gcounts, cnt_b = _geom(ctr, pts)

    wkv_t = Wqkv[DIM:, :].T
    kv_all = _kvproj(feats, wkv_t)

    nbr_kv, cfeat = _compact_gather(bitmap, gcounts, kv_all, feats, idxc_pad)

    cf = _attn_ffn(
        cfeat, nbr_kv, cnt_b,
        Wqkv[:DIM, :].T, Wout.T, bout[None, :], ln_g[None, :], ln_b[None, :],
        W1.T, b1[None, :], W2.T, b2[None, :],
    )

    feats_pad = jnp.zeros((NPAD, DIM), jnp.float32).at[:N, :].set(feats)
    out_pad = _upsample(cf, feats_pad, near8[0, :])
    return out_pad[:N, :]


# speedup vs baseline: 2.8195x; 1.0234x over previous
import functools

import numpy as np
import jax
import jax.numpy as jnp
from jax import lax
from jax.experimental import pallas as pl
from jax.experimental.pallas import tpu as pltpu
from jax.experimental.pallas import tpu_sc as plsc

DIM = 256
NUM_HEADS = 8
DH = DIM // NUM_HEADS
RADIUS = 0.3
MAX_K = 32
N = 10000
M = 2500

NPAD = 10240
MPAD = 2560
CB = 256
PB = 512
NBLK_I = MPAD // CB
NBLK_J = NPAD // PB
BMROWS = MPAD // 32
NVEC = N // 16

NTILES = 32
EROWS = NPAD // NTILES

_BIG_COORD = np.float32(1.0e9)


def _center_indices():
    perm = jax.random.permutation(jax.random.key(1234), N)
    return perm[:M].astype(jnp.int32)


def _geom_body(ctr_ref, pts_ref, bm_ref, near_ref, gc_ref, cnt_ref,
               minv, mini, run):
    i = pl.program_id(0)
    j = pl.program_id(1)
    cx = ctr_ref[:, 0:1]
    cy = ctr_ref[:, 1:2]
    cz = ctr_ref[:, 2:3]
    px = pts_ref[0:1, :]
    py = pts_ref[1:2, :]
    pz = pts_ref[2:3, :]
    dx = cx - px
    dy = cy - py
    dz = cz - pz
    d2 = (dx * dx + dy * dy) + dz * dz
    dist = jnp.sqrt(jnp.maximum(d2, 0.0))
    mask = dist < jnp.float32(RADIUS)

    rowid = lax.broadcasted_iota(jnp.int32, (CB, PB), 0)
    bits = jnp.where(mask, jnp.left_shift(jnp.int32(1), rowid & 31), 0)
    bm_ref[...] = jnp.sum(bits.reshape(CB // 32, 32, PB), axis=1)

    maskf = mask.astype(jnp.float32)
    gsel = (
        lax.broadcasted_iota(jnp.int32, (PB, PB // 16), 0) >> 4
        == lax.broadcasted_iota(jnp.int32, (PB, PB // 16), 1)
    ).astype(jnp.float32)
    gcounts = jnp.dot(maskf, gsel, preferred_element_type=jnp.float32)
    gc_ref[0] = gcounts.astype(jnp.int32)

    tile_cnt = jnp.sum(maskf, axis=1, keepdims=True)
    old = jnp.where(j == 0, 0.0, run[...])
    newrun = old + tile_cnt
    run[...] = newrun

    sl = pl.ds(j * PB, PB)
    tmin = jnp.min(dist, axis=0, keepdims=True)
    rid_g = rowid + i * CB
    tidx = jnp.min(
        jnp.where(dist == tmin, rid_g, jnp.int32(1 << 30)),
        axis=0, keepdims=True,
    )

    @pl.when(i == 0)
    def _():
        minv[0:1, sl] = tmin
        mini[0:1, sl] = tidx

    @pl.when(i > 0)
    def _():
        upd = tmin < minv[0:1, sl]
        mini[0:1, sl] = jnp.where(upd, tidx, mini[0:1, sl])
        minv[0:1, sl] = jnp.where(upd, tmin, minv[0:1, sl])

    @pl.when(i == NBLK_I - 1)
    def _():
        near_ref[...] = jnp.broadcast_to(mini[0:1, sl], (8, PB))

    @pl.when(j == NBLK_J - 1)
    def _():
        cnt = jnp.minimum(newrun, np.float32(MAX_K)).astype(jnp.int32)
        cnt_ref[...] = jnp.broadcast_to(cnt, (CB, 128))


def _geom(ctr, pts, *, interpret=False):
    return pl.pallas_call(
        _geom_body,
        grid=(NBLK_I, NBLK_J),
        in_specs=[
            pl.BlockSpec((CB, 8), lambda i, j: (i, 0)),
            pl.BlockSpec((8, PB), lambda i, j: (0, j)),
        ],
        out_specs=[
            pl.BlockSpec((CB // 32, PB), lambda i, j: (i, j)),
            pl.BlockSpec((8, PB), lambda i, j: (0, j)),
            pl.BlockSpec((1, CB, PB // 16), lambda i, j: (j, i, 0)),
            pl.BlockSpec((CB, 128), lambda i, j: (i, 0)),
        ],
        out_shape=[
            jax.ShapeDtypeStruct((BMROWS, NPAD), jnp.int32),
            jax.ShapeDtypeStruct((8, NPAD), jnp.int32),
            jax.ShapeDtypeStruct((NBLK_J, MPAD, PB // 16), jnp.int32),
            jax.ShapeDtypeStruct((MPAD, 128), jnp.int32),
        ],
        scratch_shapes=[
            pltpu.VMEM((8, NPAD), jnp.float32),
            pltpu.VMEM((8, NPAD), jnp.int32),
            pltpu.VMEM((CB, 1), jnp.float32),
        ],
        interpret=interpret,
    )(ctr, pts)


def _kvproj_body(f_ref, w_ref, o_ref):
    o_ref[...] = jnp.dot(f_ref[...], w_ref[...],
                         preferred_element_type=jnp.float32)


def _kvproj(feats, wkv_t, *, interpret=False):
    rb = 1000
    return pl.pallas_call(
        _kvproj_body,
        grid=(N // rb,),
        in_specs=[
            pl.BlockSpec((rb, DIM), lambda i: (i, 0)),
            pl.BlockSpec((DIM, 2 * DIM), lambda i: (0, 0)),
        ],
        out_specs=pl.BlockSpec((rb, 2 * DIM), lambda i: (i, 0)),
        out_shape=jax.ShapeDtypeStruct((N, 2 * DIM), jnp.float32),
        interpret=interpret,
    )(feats, wkv_t)


QUOTA = MPAD // NTILES
GCW = NBLK_J * 32


def _compact_body(bm_hbm, gc_hbm, kv_hbm, feats_hbm, idxc_hbm,
                  nbrkv_hbm, cfeat_hbm,
                  bmslab, gcbuf, idxbuf, idxout0, idxout1, kvbuf, icvec,
                  cfbuf, nsm, semgc, semg, semw, semf):
    wid = lax.axis_index("s") * 2 + lax.axis_index("c")
    base = wid * QUOTA
    r0 = base >> 5
    rlast = (base + QUOTA - 1) >> 5
    zeros16 = jnp.zeros((16,), jnp.int32)
    iota16 = lax.iota(jnp.int32, 16)

    for rr in range(4):
        @pl.when(r0 + rr <= rlast)
        def _():
            pltpu.sync_copy(bm_hbm.at[r0 + rr],
                            bmslab.at[pl.ds(rr * NPAD, NPAD)])

    pltpu.sync_copy(idxc_hbm.at[pl.ds(base, QUOTA)], icvec)
    pltpu.async_copy(feats_hbm.at[icvec], cfbuf, semf).wait()
    pltpu.sync_copy(cfbuf, cfeat_hbm.at[pl.ds(base, QUOTA)])

    pltpu.async_copy(gc_hbm.at[base], gcbuf.at[pl.ds(0, GCW)], semgc)

    def do_center(c, slot):
        gc_s = gcbuf.at[pl.ds(slot * GCW, GCW)]
        gc_o = gcbuf.at[pl.ds((1 - slot) * GCW, GCW)]
        io_s = idxout0 if slot == 0 else idxout1
        io_o = idxout1 if slot == 0 else idxout0
        kv_s = kvbuf.at[slot]
        kv_o = kvbuf.at[1 - slot]

        @pl.when(c < M)
        def _():
            @pl.when(jnp.logical_and(c + 1 < M, c + 1 - base < QUOTA))
            def _():
                pltpu.async_copy(gc_hbm.at[c + 1], gc_o, semgc)

            pltpu.make_async_copy(gc_hbm.at[c], gc_s, semgc).wait()

            idxbuf[pl.ds(0, 16)] = zeros16
            idxbuf[pl.ds(16, 16)] = zeros16
            idxbuf[pl.ds(32, 16)] = zeros16
            nsm[0] = jnp.int32(0)
            rowoff = ((c >> 5) - r0) * NPAD
            cbit = c & 31

            def chunk_body(ch, _):
                n0 = nsm[0]

                @pl.when(n0 < MAX_K)
                def _():
                    g1 = gcbuf[pl.ds(slot * GCW + ch * 32, 16)]
                    g2 = gcbuf[pl.ds(slot * GCW + ch * 32 + 16, 16)]
                    n = n0
                    for k in range(32):
                        v = ch * 32 + k
                        gk = g1[k] if k < 16 else g2[k - 16]

                        @pl.when(gk > 0)
                        def _(n=n, v=v):
                            words = bmslab[pl.ds(rowoff + v * 16, 16)]
                            mvec = (jnp.right_shift(words, cbit) & 1) == 1
                            vals = iota16 + v * 16
                            off = jnp.minimum(n, MAX_K)
                            plsc.store_compressed(
                                idxbuf.at[pl.ds(off, 16)], vals, mask=mvec)

                        n = n + gk
                    nsm[0] = n
                return ()

            lax.fori_loop(0, NBLK_J, chunk_body, ())

            io_s[pl.ds(0, 16)] = idxbuf[pl.ds(0, 16)]
            io_s[pl.ds(16, 16)] = idxbuf[pl.ds(16, 16)]

            pltpu.async_copy(kv_hbm.at[io_s], kv_s, semg).wait()
            pltpu.sync_copy(kv_s, nbrkv_hbm.at[c])

    def center_body(t, _):
        c0 = base + 2 * t
        do_center(c0, 0)
        do_center(c0 + 1, 1)
        return ()

    lax.fori_loop(0, QUOTA // 2, center_body, ())


def _compact_gather(bitmap, gcounts, kv_all, feats, idxc_pad):
    mesh = plsc.VectorSubcoreMesh(core_axis_name="c", subcore_axis_name="s")
    f = functools.partial(
        pl.kernel,
        mesh=mesh,
        compiler_params=pltpu.CompilerParams(needs_layout_passes=False),
        out_type=[
            jax.ShapeDtypeStruct((MPAD, MAX_K, 2 * DIM), jnp.float32),
            jax.ShapeDtypeStruct((MPAD, DIM), jnp.float32),
        ],
        scratch_types=[
            pltpu.VMEM((4 * NPAD,), jnp.int32),
            pltpu.VMEM((2 * GCW,), jnp.int32),
            pltpu.VMEM((48,), jnp.int32),
            pltpu.VMEM((MAX_K,), jnp.int32),
            pltpu.VMEM((MAX_K,), jnp.int32),
            pltpu.VMEM((2, MAX_K, 2 * DIM), jnp.float32),
            pltpu.VMEM((QUOTA,), jnp.int32),
            pltpu.VMEM((QUOTA, DIM), jnp.float32),
            pltpu.SMEM((1,), jnp.int32),
            pltpu.SemaphoreType.DMA,
            pltpu.SemaphoreType.DMA,
            pltpu.SemaphoreType.DMA,
            pltpu.SemaphoreType.DMA,
        ],
    )(_compact_body)
    return f(bitmap, gcounts, kv_all, feats, idxc_pad)


def _ln(x, g, b):
    mu = jnp.mean(x, axis=-1, keepdims=True)
    var = jnp.mean((x - mu) ** 2, axis=-1, keepdims=True)
    return (x - mu) / jnp.sqrt(var + 1e-5) * g + b


CB2 = 128


def _attn_body(cf_ref, kv_ref, cnt_ref, wq_ref, wo_ref, bo_ref,
               g_ref, b_ref, w1_ref, b1_ref, w2_ref, b2_ref, out_ref):
    scale = np.float32(DH ** -0.5)
    cf0 = cf_ref[...]
    q = jnp.dot(cf0, wq_ref[...], preferred_element_type=jnp.float32) * scale
    cnt = cnt_ref[:, 0:1]
    valid = lax.broadcasted_iota(jnp.int32, (CB2, MAX_K), 1) < cnt

    outs = []
    for h in range(NUM_HEADS):
        kh = kv_ref[:, :, h * DH:(h + 1) * DH]
        vh = kv_ref[:, :, DIM + h * DH:DIM + (h + 1) * DH]
        qh = q[:, None, h * DH:(h + 1) * DH]
        s = jnp.sum(qh * kh, axis=-1)
        s = jnp.where(valid, s, jnp.float32(-1e9))
        m = jnp.max(s, axis=-1, keepdims=True)
        e = jnp.exp(s - m)
        a = e / jnp.sum(e, axis=-1, keepdims=True)
        outs.append(jnp.sum(a[:, :, None] * vh, axis=1))
    att = jnp.concatenate(outs, axis=-1)

    upd = jnp.dot(att, wo_ref[...],
                  preferred_element_type=jnp.float32) + bo_ref[...]
    g = g_ref[...]
    b = b_ref[...]
    cf1 = cf0 + _ln(upd, g, b)
    h1 = jnp.maximum(
        jnp.dot(cf1, w1_ref[...], preferred_element_type=jnp.float32)
        + b1_ref[...], 0.0)
    f2 = jnp.dot(h1, w2_ref[...],
                 preferred_element_type=jnp.float32) + b2_ref[...]
    out_ref[...] = cf1 + _ln(f2, g, b)


def _attn_ffn(cfeat, nbr_kv, cnt_b, wq_t, wo_t, bout, ln_g, ln_b,
              w1_t, b1, w2_t, b2, *, interpret=False):
    return pl.pallas_call(
        _attn_body,
        grid=(MPAD // CB2,),
        in_specs=[
            pl.BlockSpec((CB2, DIM), lambda i: (i, 0)),
            pl.BlockSpec((CB2, MAX_K, 2 * DIM), lambda i: (i, 0, 0)),
            pl.BlockSpec((CB2, 128), lambda i: (i, 0)),
            pl.BlockSpec((DIM, DIM), lambda i: (0, 0)),
            pl.BlockSpec((DIM, DIM), lambda i: (0, 0)),
            pl.BlockSpec((1, DIM), lambda i: (0, 0)),
            pl.BlockSpec((1, DIM), lambda i: (0, 0)),
            pl.BlockSpec((1, DIM), lambda i: (0, 0)),
            pl.BlockSpec((DIM, 4 * DIM), lambda i: (0, 0)),
            pl.BlockSpec((1, 4 * DIM), lambda i: (0, 0)),
            pl.BlockSpec((4 * DIM, DIM), lambda i: (0, 0)),
            pl.BlockSpec((1, DIM), lambda i: (0, 0)),
        ],
        out_specs=pl.BlockSpec((CB2, DIM), lambda i: (i, 0)),
        out_shape=jax.ShapeDtypeStruct((MPAD, DIM), jnp.float32),
        interpret=interpret,
    )(cfeat, nbr_kv, cnt_b, wq_t, wo_t, bout, ln_g, ln_b, w1_t, b1, w2_t, b2)


def _ups_body(cf_hbm, feats_hbm, near_hbm, out_hbm,
              idxv, cbuf, fbuf, obuf, sem):
    wid = lax.axis_index("s") * 2 + lax.axis_index("c")
    base = wid * EROWS

    def chunk_body(t, _):
        start = base + t * 32
        pltpu.sync_copy(near_hbm.at[pl.ds(start, 32)], idxv)
        cp = pltpu.async_copy(cf_hbm.at[idxv], cbuf, sem)
        pltpu.sync_copy(feats_hbm.at[pl.ds(start, 32)], fbuf)
        cp.wait()

        def row_body(i, _):
            def lane_body(kk, _):
                sl = pl.ds(kk * 16, 16)
                obuf[i, sl] = fbuf[i, sl] + cbuf[i, sl]
                return ()
            lax.fori_loop(0, DIM // 16, lane_body, ())
            return ()

        lax.fori_loop(0, 32, row_body, ())
        pltpu.sync_copy(obuf, out_hbm.at[pl.ds(start, 32)])
        return ()

    lax.fori_loop(0, EROWS // 32, chunk_body, ())


def _upsample(cf, feats_pad, nearest):
    mesh = plsc.VectorSubcoreMesh(core_axis_name="c", subcore_axis_name="s")
    f = functools.partial(
        pl.kernel,
        mesh=mesh,
        compiler_params=pltpu.CompilerParams(needs_layout_passes=False),
        out_type=jax.ShapeDtypeStruct((NPAD, DIM), jnp.float32),
        scratch_types=[
            pltpu.VMEM((32,), jnp.int32),
            pltpu.VMEM((32, DIM), jnp.float32),
            pltpu.VMEM((32, DIM), jnp.float32),
            pltpu.VMEM((32, DIM), jnp.float32),
            pltpu.SemaphoreType.DMA,
        ],
    )(_ups_body)
    return f(cf, feats_pad, nearest)


def kernel(xyz, feats, Wqkv, Wout, bout, ln_g, ln_b, W1, b1, W2, b2):
    idxc = _center_indices()
    idxc_pad = jnp.zeros((MPAD,), jnp.int32).at[:M].set(idxc)

    center_xyz = xyz[idxc]
    ctr = jnp.full((MPAD, 8), _BIG_COORD, jnp.float32)
    ctr = ctr.at[:M, 0:3].set(center_xyz)
    pts = jnp.full((8, NPAD), _BIG_COORD, jnp.float32)
    pts = pts.at[0:3, :N].set(xyz.T)

    bitmap, near8, gcounts, cnt_b = _geom(ctr, pts)

    wkv_t = Wqkv[DIM:, :].T
    kv_all = _kvproj(feats, wkv_t)

    gc_c = jnp.transpose(gcounts, (1, 0, 2)).reshape(MPAD, GCW)
    nbr_kv, cfeat = _compact_gather(bitmap, gc_c, kv_all, feats, idxc_pad)

    cf = _attn_ffn(
        cfeat, nbr_kv, cnt_b,
        Wqkv[:DIM, :].T, Wout.T, bout[None, :], ln_g[None, :], ln_b[None, :],
        W1.T, b1[None, :], W2.T, b2[None, :],
    )

    feats_pad = jnp.zeros((NPAD, DIM), jnp.float32).at[:N, :].set(feats)
    out_pad = _upsample(cf, feats_pad, near8[0, :])
    return out_pad[:N, :]


# speedup vs baseline: 2.8318x; 1.0044x over previous
import functools

import numpy as np
import jax
import jax.numpy as jnp
from jax import lax
from jax.experimental import pallas as pl
from jax.experimental.pallas import tpu as pltpu
from jax.experimental.pallas import tpu_sc as plsc

DIM = 256
NUM_HEADS = 8
DH = DIM // NUM_HEADS
RADIUS = 0.3
MAX_K = 32
N = 10000
M = 2500

NPAD = 10240
MPAD = 2560
CB = 256
PB = 512
NBLK_I = MPAD // CB
NBLK_J = NPAD // PB
BMROWS = MPAD // 32
NVEC = N // 16

NTILES = 32
EROWS = NPAD // NTILES

_BIG_COORD = np.float32(1.0e9)


def _center_indices():
    perm = jax.random.permutation(jax.random.key(1234), N)
    return perm[:M].astype(jnp.int32)


def _geom_body(ctr_ref, pts_ref, bm_ref, near_ref, gc_ref, cnt_ref,
               minv, mini, run):
    i = pl.program_id(0)
    j = pl.program_id(1)
    cx = ctr_ref[:, 0:1]
    cy = ctr_ref[:, 1:2]
    cz = ctr_ref[:, 2:3]
    px = pts_ref[0:1, :]
    py = pts_ref[1:2, :]
    pz = pts_ref[2:3, :]
    dx = cx - px
    dy = cy - py
    dz = cz - pz
    d2 = (dx * dx + dy * dy) + dz * dz
    dist = jnp.sqrt(jnp.maximum(d2, 0.0))
    mask = dist < jnp.float32(RADIUS)

    rowid = lax.broadcasted_iota(jnp.int32, (CB, PB), 0)
    bits = jnp.where(mask, jnp.left_shift(jnp.int32(1), rowid & 31), 0)
    bm_ref[...] = jnp.sum(bits.reshape(CB // 32, 32, PB), axis=1)

    maskf = mask.astype(jnp.float32)
    gsel = (
        lax.broadcasted_iota(jnp.int32, (PB, PB // 16), 0) >> 4
        == lax.broadcasted_iota(jnp.int32, (PB, PB // 16), 1)
    ).astype(jnp.float32)
    gcounts = jnp.dot(maskf, gsel, preferred_element_type=jnp.float32)
    gc_ref[0] = gcounts.astype(jnp.int32)

    tile_cnt = jnp.sum(maskf, axis=1, keepdims=True)
    old = jnp.where(j == 0, 0.0, run[...])
    newrun = old + tile_cnt
    run[...] = newrun

    sl = pl.ds(j * PB, PB)
    tmin = jnp.min(dist, axis=0, keepdims=True)
    rid_g = rowid + i * CB
    tidx = jnp.min(
        jnp.where(dist == tmin, rid_g, jnp.int32(1 << 30)),
        axis=0, keepdims=True,
    )

    @pl.when(i == 0)
    def _():
        minv[0:1, sl] = tmin
        mini[0:1, sl] = tidx

    @pl.when(i > 0)
    def _():
        upd = tmin < minv[0:1, sl]
        mini[0:1, sl] = jnp.where(upd, tidx, mini[0:1, sl])
        minv[0:1, sl] = jnp.where(upd, tmin, minv[0:1, sl])

    @pl.when(i == NBLK_I - 1)
    def _():
        near_ref[...] = jnp.broadcast_to(mini[0:1, sl], (8, PB))

    @pl.when(j == NBLK_J - 1)
    def _():
        cnt = jnp.minimum(newrun, np.float32(MAX_K)).astype(jnp.int32)
        cnt_ref[...] = jnp.broadcast_to(cnt, (CB, 128))


def _geom(ctr, pts, *, interpret=False):
    return pl.pallas_call(
        _geom_body,
        grid=(NBLK_I, NBLK_J),
        in_specs=[
            pl.BlockSpec((CB, 8), lambda i, j: (i, 0)),
            pl.BlockSpec((8, PB), lambda i, j: (0, j)),
        ],
        out_specs=[
            pl.BlockSpec((CB // 32, PB), lambda i, j: (i, j)),
            pl.BlockSpec((8, PB), lambda i, j: (0, j)),
            pl.BlockSpec((1, CB, PB // 16), lambda i, j: (j, i, 0)),
            pl.BlockSpec((CB, 128), lambda i, j: (i, 0)),
        ],
        out_shape=[
            jax.ShapeDtypeStruct((BMROWS, NPAD), jnp.int32),
            jax.ShapeDtypeStruct((8, NPAD), jnp.int32),
            jax.ShapeDtypeStruct((NBLK_J, MPAD, PB // 16), jnp.int32),
            jax.ShapeDtypeStruct((MPAD, 128), jnp.int32),
        ],
        scratch_shapes=[
            pltpu.VMEM((8, NPAD), jnp.float32),
            pltpu.VMEM((8, NPAD), jnp.int32),
            pltpu.VMEM((CB, 1), jnp.float32),
        ],
        interpret=interpret,
    )(ctr, pts)


def _kvproj_body(f_ref, w_ref, o_ref):
    o_ref[...] = jnp.dot(f_ref[...], w_ref[...],
                         preferred_element_type=jnp.float32)


def _kvproj(feats, wkv_t, *, interpret=False):
    rb = 1000
    return pl.pallas_call(
        _kvproj_body,
        grid=(N // rb,),
        in_specs=[
            pl.BlockSpec((rb, DIM), lambda i: (i, 0)),
            pl.BlockSpec((DIM, 2 * DIM), lambda i: (0, 0)),
        ],
        out_specs=pl.BlockSpec((rb, 2 * DIM), lambda i: (i, 0)),
        out_shape=jax.ShapeDtypeStruct((N, 2 * DIM), jnp.float32),
        interpret=interpret,
    )(feats, wkv_t)


QUOTA = MPAD // NTILES
GCW = NBLK_J * 32


def _compact_body(bm_hbm, gc_hbm, kv_hbm, feats_hbm, idxc_hbm,
                  nbrkv_hbm, cfeat_hbm,
                  bmslab, gcbuf, idxbuf, idxout0, idxout1, kvbuf, icvec,
                  cfbuf, nsm, semgc, semg, semw, semf):
    wid = lax.axis_index("s") * 2 + lax.axis_index("c")
    base = wid * QUOTA
    r0 = base >> 5
    rlast = (base + QUOTA - 1) >> 5
    zeros16 = jnp.zeros((16,), jnp.int32)
    iota16 = lax.iota(jnp.int32, 16)

    for rr in range(4):
        @pl.when(r0 + rr <= rlast)
        def _():
            pltpu.sync_copy(bm_hbm.at[r0 + rr],
                            bmslab.at[pl.ds(rr * NPAD, NPAD)])

    pltpu.sync_copy(idxc_hbm.at[pl.ds(base, QUOTA)], icvec)
    pltpu.async_copy(feats_hbm.at[icvec], cfbuf, semf).wait()
    pltpu.sync_copy(cfbuf, cfeat_hbm.at[pl.ds(base, QUOTA)])

    pltpu.async_copy(gc_hbm.at[base], gcbuf.at[pl.ds(0, GCW)], semgc)

    def do_center(c, slot):
        gc_s = gcbuf.at[pl.ds(slot * GCW, GCW)]
        gc_o = gcbuf.at[pl.ds((1 - slot) * GCW, GCW)]
        io_s = idxout0 if slot == 0 else idxout1
        io_o = idxout1 if slot == 0 else idxout0
        kv_s = kvbuf.at[slot]
        kv_o = kvbuf.at[1 - slot]

        @pl.when(c < M)
        def _():
            @pl.when(jnp.logical_and(c + 1 < M, c + 1 - base < QUOTA))
            def _():
                pltpu.async_copy(gc_hbm.at[c + 1], gc_o, semgc)

            pltpu.make_async_copy(gc_hbm.at[c], gc_s, semgc).wait()

            idxbuf[pl.ds(0, 16)] = zeros16
            idxbuf[pl.ds(16, 16)] = zeros16
            idxbuf[pl.ds(32, 16)] = zeros16
            nsm[0] = jnp.int32(0)
            rowoff = ((c >> 5) - r0) * NPAD
            cbit = c & 31

            def chunk_body(ch, _):
                n0 = nsm[0]

                @pl.when(n0 < MAX_K)
                def _():
                    g1 = gcbuf[pl.ds(slot * GCW + ch * 32, 16)]
                    g2 = gcbuf[pl.ds(slot * GCW + ch * 32 + 16, 16)]
                    n = n0
                    for k in range(32):
                        v = ch * 32 + k
                        gk = g1[k] if k < 16 else g2[k - 16]

                        @pl.when(gk > 0)
                        def _(n=n, v=v):
                            words = bmslab[pl.ds(rowoff + v * 16, 16)]
                            mvec = (jnp.right_shift(words, cbit) & 1) == 1
                            vals = iota16 + v * 16
                            off = jnp.minimum(n, MAX_K)
                            plsc.store_compressed(
                                idxbuf.at[pl.ds(off, 16)], vals, mask=mvec)

                        n = n + gk
                    nsm[0] = n
                return ()

            lax.fori_loop(0, NBLK_J, chunk_body, ())

            io_s[pl.ds(0, 16)] = idxbuf[pl.ds(0, 16)]
            io_s[pl.ds(16, 16)] = idxbuf[pl.ds(16, 16)]

            @pl.when(c - base >= 2)
            def _():
                pltpu.make_async_copy(kv_s, nbrkv_hbm.at[c - 2],
                                      semw).wait()

            @pl.when(c - base >= 1)
            def _():
                pltpu.make_async_copy(kv_hbm.at[io_o], kv_o, semg).wait()
                pltpu.async_copy(kv_o, nbrkv_hbm.at[c - 1], semw)

            pltpu.async_copy(kv_hbm.at[io_s], kv_s, semg)

    def center_body(t, _):
        c0 = base + 2 * t
        do_center(c0, 0)
        do_center(c0 + 1, 1)
        return ()

    lax.fori_loop(0, QUOTA // 2, center_body, ())

    nreal = jnp.minimum(M - base, QUOTA)
    cl = base + nreal - 1

    def drain(slot):
        kv_s = kvbuf.at[slot]
        kv_o = kvbuf.at[1 - slot]
        io_s = idxout0 if slot == 0 else idxout1
        pltpu.make_async_copy(kv_hbm.at[io_s], kv_s, semg).wait()
        pltpu.sync_copy(kv_s, nbrkv_hbm.at[cl])

        @pl.when(nreal >= 2)
        def _():
            pltpu.make_async_copy(kv_o, nbrkv_hbm.at[cl - 1], semw).wait()

    @pl.when((nreal - 1) % 2 == 0)
    def _():
        drain(0)

    @pl.when((nreal - 1) % 2 == 1)
    def _():
        drain(1)


def _compact_gather(bitmap, gcounts, kv_all, feats, idxc_pad):
    mesh = plsc.VectorSubcoreMesh(core_axis_name="c", subcore_axis_name="s")
    f = functools.partial(
        pl.kernel,
        mesh=mesh,
        compiler_params=pltpu.CompilerParams(needs_layout_passes=False),
        out_type=[
            jax.ShapeDtypeStruct((MPAD, MAX_K, 2 * DIM), jnp.float32),
            jax.ShapeDtypeStruct((MPAD, DIM), jnp.float32),
        ],
        scratch_types=[
            pltpu.VMEM((4 * NPAD,), jnp.int32),
            pltpu.VMEM((2 * GCW,), jnp.int32),
            pltpu.VMEM((48,), jnp.int32),
            pltpu.VMEM((MAX_K,), jnp.int32),
            pltpu.VMEM((MAX_K,), jnp.int32),
            pltpu.VMEM((2, MAX_K, 2 * DIM), jnp.float32),
            pltpu.VMEM((QUOTA,), jnp.int32),
            pltpu.VMEM((QUOTA, DIM), jnp.float32),
            pltpu.SMEM((1,), jnp.int32),
            pltpu.SemaphoreType.DMA,
            pltpu.SemaphoreType.DMA,
            pltpu.SemaphoreType.DMA,
            pltpu.SemaphoreType.DMA,
        ],
    )(_compact_body)
    return f(bitmap, gcounts, kv_all, feats, idxc_pad)


def _ln(x, g, b):
    mu = jnp.mean(x, axis=-1, keepdims=True)
    var = jnp.mean((x - mu) ** 2, axis=-1, keepdims=True)
    return (x - mu) / jnp.sqrt(var + 1e-5) * g + b


CB2 = 128


def _attn_body(cf_ref, kv_ref, cnt_ref, wq_ref, wo_ref, bo_ref,
               g_ref, b_ref, w1_ref, b1_ref, w2_ref, b2_ref, out_ref):
    scale = np.float32(DH ** -0.5)
    cf0 = cf_ref[...]
    q = jnp.dot(cf0, wq_ref[...], preferred_element_type=jnp.float32) * scale
    cnt = cnt_ref[:, 0:1]
    valid = lax.broadcasted_iota(jnp.int32, (CB2, MAX_K), 1) < cnt

    outs = []
    for h in range(NUM_HEADS):
        kh = kv_ref[:, :, h * DH:(h + 1) * DH]
        vh = kv_ref[:, :, DIM + h * DH:DIM + (h + 1) * DH]
        qh = q[:, None, h * DH:(h + 1) * DH]
        s = jnp.sum(qh * kh, axis=-1)
        s = jnp.where(valid, s, jnp.float32(-1e9))
        m = jnp.max(s, axis=-1, keepdims=True)
        e = jnp.exp(s - m)
        a = e / jnp.sum(e, axis=-1, keepdims=True)
        outs.append(jnp.sum(a[:, :, None] * vh, axis=1))
    att = jnp.concatenate(outs, axis=-1)

    upd = jnp.dot(att, wo_ref[...],
                  preferred_element_type=jnp.float32) + bo_ref[...]
    g = g_ref[...]
    b = b_ref[...]
    cf1 = cf0 + _ln(upd, g, b)
    h1 = jnp.maximum(
        jnp.dot(cf1, w1_ref[...], preferred_element_type=jnp.float32)
        + b1_ref[...], 0.0)
    f2 = jnp.dot(h1, w2_ref[...],
                 preferred_element_type=jnp.float32) + b2_ref[...]
    out_ref[...] = cf1 + _ln(f2, g, b)


def _attn_ffn(cfeat, nbr_kv, cnt_b, wq_t, wo_t, bout, ln_g, ln_b,
              w1_t, b1, w2_t, b2, *, interpret=False):
    return pl.pallas_call(
        _attn_body,
        grid=(MPAD // CB2,),
        in_specs=[
            pl.BlockSpec((CB2, DIM), lambda i: (i, 0)),
            pl.BlockSpec((CB2, MAX_K, 2 * DIM), lambda i: (i, 0, 0)),
            pl.BlockSpec((CB2, 128), lambda i: (i, 0)),
            pl.BlockSpec((DIM, DIM), lambda i: (0, 0)),
            pl.BlockSpec((DIM, DIM), lambda i: (0, 0)),
            pl.BlockSpec((1, DIM), lambda i: (0, 0)),
            pl.BlockSpec((1, DIM), lambda i: (0, 0)),
            pl.BlockSpec((1, DIM), lambda i: (0, 0)),
            pl.BlockSpec((DIM, 4 * DIM), lambda i: (0, 0)),
            pl.BlockSpec((1, 4 * DIM), lambda i: (0, 0)),
            pl.BlockSpec((4 * DIM, DIM), lambda i: (0, 0)),
            pl.BlockSpec((1, DIM), lambda i: (0, 0)),
        ],
        out_specs=pl.BlockSpec((CB2, DIM), lambda i: (i, 0)),
        out_shape=jax.ShapeDtypeStruct((MPAD, DIM), jnp.float32),
        interpret=interpret,
    )(cfeat, nbr_kv, cnt_b, wq_t, wo_t, bout, ln_g, ln_b, w1_t, b1, w2_t, b2)


def _ups_body(cf_hbm, feats_hbm, near_hbm, out_hbm,
              idxv, cbuf, fbuf, obuf, sem):
    wid = lax.axis_index("s") * 2 + lax.axis_index("c")
    base = wid * EROWS

    def chunk_body(t, _):
        start = base + t * 32
        pltpu.sync_copy(near_hbm.at[pl.ds(start, 32)], idxv)
        cp = pltpu.async_copy(cf_hbm.at[idxv], cbuf, sem)
        pltpu.sync_copy(feats_hbm.at[pl.ds(start, 32)], fbuf)
        cp.wait()

        def row_body(i, _):
            def lane_body(kk, _):
                sl = pl.ds(kk * 16, 16)
                obuf[i, sl] = fbuf[i, sl] + cbuf[i, sl]
                return ()
            lax.fori_loop(0, DIM // 16, lane_body, ())
            return ()

        lax.fori_loop(0, 32, row_body, ())
        pltpu.sync_copy(obuf, out_hbm.at[pl.ds(start, 32)])
        return ()

    lax.fori_loop(0, EROWS // 32, chunk_body, ())


def _upsample(cf, feats_pad, nearest):
    mesh = plsc.VectorSubcoreMesh(core_axis_name="c", subcore_axis_name="s")
    f = functools.partial(
        pl.kernel,
        mesh=mesh,
        compiler_params=pltpu.CompilerParams(needs_layout_passes=False),
        out_type=jax.ShapeDtypeStruct((NPAD, DIM), jnp.float32),
        scratch_types=[
            pltpu.VMEM((32,), jnp.int32),
            pltpu.VMEM((32, DIM), jnp.float32),
            pltpu.VMEM((32, DIM), jnp.float32),
            pltpu.VMEM((32, DIM), jnp.float32),
            pltpu.SemaphoreType.DMA,
        ],
    )(_ups_body)
    return f(cf, feats_pad, nearest)


def kernel(xyz, feats, Wqkv, Wout, bout, ln_g, ln_b, W1, b1, W2, b2):
    idxc = _center_indices()
    idxc_pad = jnp.zeros((MPAD,), jnp.int32).at[:M].set(idxc)

    center_xyz = xyz[idxc]
    ctr = jnp.full((MPAD, 8), _BIG_COORD, jnp.float32)
    ctr = ctr.at[:M, 0:3].set(center_xyz)
    pts = jnp.full((8, NPAD), _BIG_COORD, jnp.float32)
    pts = pts.at[0:3, :N].set(xyz.T)

    bitmap, near8, gcounts, cnt_b = _geom(ctr, pts)

    wkv_t = Wqkv[DIM:, :].T
    kv_all = _kvproj(feats, wkv_t)

    gc_c = jnp.transpose(gcounts, (1, 0, 2)).reshape(MPAD, GCW)
    nbr_kv, cfeat = _compact_gather(bitmap, gc_c, kv_all, feats, idxc_pad)

    cf = _attn_ffn(
        cfeat, nbr_kv, cnt_b,
        Wqkv[:DIM, :].T, Wout.T, bout[None, :], ln_g[None, :], ln_b[None, :],
        W1.T, b1[None, :], W2.T, b2[None, :],
    )

    feats_pad = jnp.zeros((NPAD, DIM), jnp.float32).at[:N, :].set(feats)
    out_pad = _upsample(cf, feats_pad, near8[0, :])
    return out_pad[:N, :]
